# baseline (speedup 1.0000x reference)
import math

import jax
import jax.numpy as jnp
from jax import lax
from jax.experimental import pallas as pl
from jax.experimental.pallas import tpu as pltpu
from jax.experimental.pallas import tpu_sc as plsc

VSZ = 1000000
DSZ = 64
MXLEN = 1000
MAX_TIMESCALE = 10000.0
B = 1024
L = 200

NC = 2
NS = 16
NW = NC * NS
BG = 128
NBG = B // BG
NBLK = L * NBG
PER_W = NBLK // NW
SCALE = math.sqrt(DSZ)
NG = BG // 16
NCV = DSZ // 16

FMT_B = 128
HALF_V = 500096
FMT_G = HALF_V // FMT_B
NBK = (VSZ + 127) // 128
FMT_IT = (NBK + NW - 1) // NW
FMT_LOOP = ((FMT_IT + 2) + 3) // 4 * 4 + 4


def _pos_encoding():
    log_inc = math.log(MAX_TIMESCALE) / DSZ
    inv = jnp.exp(jnp.arange(0, DSZ, 2, dtype=jnp.float32) * -log_inc)
    pos = jnp.arange(0, MXLEN, dtype=jnp.float32)[:, None]
    pe = jnp.zeros((MXLEN, DSZ), jnp.float32)
    pe = pe.at[:, 0::2].set(jnp.sin(pos * inv))
    pe = pe.at[:, 1::2].set(jnp.cos(pos * inv))
    return pe[:L]


def _tc_fmt_body(a_ref, b_ref, o_ref):
    o_ref[...] = jnp.concatenate([a_ref[...].T, b_ref[...].T], axis=1)


def _fmt_body(tblT, scratch, sbuf, tbuf, *sems):
    rs = sems[:4]
    ws = sems[4:]
    wid = lax.axis_index("s") * NC + lax.axis_index("c")
    iota = lax.iota(jnp.int32, 16)
    rds = [iota + 16 * d0 for d0 in range(4)]

    def blk_of(i):
        return wid + NW * i

    def fire_read(i, b):
        blk = blk_of(i)

        @pl.when(blk < NBK)
        def _():
            pltpu.async_copy(tblT.at[:, pl.ds(blk * 128, 128)], sbuf.at[b],
                             rs[b])

    def wait_read(i, b):
        @pl.when(blk_of(i) < NBK)
        def _():
            pltpu.make_async_copy(tblT.at[:, pl.ds(0, 128)], sbuf.at[b],
                                  rs[b]).wait()

    def fire_write(i, b):
        blk = blk_of(i)

        @pl.when(blk < NBK - 1)
        def _():
            pltpu.async_copy(tbuf.at[b], scratch.at[pl.ds(blk * 64, 64), :],
                             ws[b])

        @pl.when(blk == NBK - 1)
        def _():
            pltpu.async_copy(tbuf.at[b, pl.ds(0, 32), :],
                             scratch.at[pl.ds(blk * 64, 32), :], ws[b])

    def wait_write(i, b):
        blk = blk_of(i)

        @pl.when(blk < NBK - 1)
        def _():
            pltpu.make_async_copy(tbuf.at[b],
                                  scratch.at[pl.ds(0, 64), :], ws[b]).wait()

        @pl.when(blk == NBK - 1)
        def _():
            pltpu.make_async_copy(tbuf.at[b, pl.ds(0, 32), :],
                                  scratch.at[pl.ds(0, 32), :], ws[b]).wait()

    def transpose(i, b):
        @pl.when(blk_of(i) < NBK)
        def _():
            sb = sbuf.at[b]
            tb = tbuf.at[b]

            @pl.loop(0, 128, unroll=2)
            def _j(j0):
                cj = (jnp.full((16,), j0, jnp.int32) + iota) & 127
                pr = lax.shift_right_logical(cj, 1)
                pc0 = (cj & 1) * DSZ
                for d0 in range(4):
                    vals = plsc.load_gather(sb, [rds[d0], cj])
                    plsc.store_scatter(tb, [pr, pc0 + rds[d0]], vals)

    fire_read(0, 0)
    fire_read(1, 1)

    @pl.loop(0, FMT_LOOP, step=4)
    def _grp(i):
        for b in range(4):
            ii = i + b
            wait_read(ii, b)
            b2 = (b + 2) % 4

            @pl.when(ii >= 2)
            def _():
                wait_write(ii - 2, b2)

            fire_read(ii + 2, b2)
            transpose(ii, b)
            fire_write(ii, b)


def _sc_body(xT, pe_hbm, tbl2, out_jm,
             pe_v, xl_v, idxraw_v, idx2_v, gbuf, obuf, *sems):
    gs = sems[:2]
    ws = sems[2:]
    wid = lax.axis_index("s") * NC + lax.axis_index("c")
    base = wid * PER_W

    pltpu.sync_copy(pe_hbm, pe_v)
    l0 = pl.multiple_of(jnp.minimum((base >> 3) & ~7, L - 16), 8)
    pltpu.sync_copy(xT.at[pl.ds(l0, 16), :], xl_v)

    def prep(blk, buf):
        l = blk >> 3
        bg = blk & 7
        for g in range(NG):
            sl = pl.ds(16 * g, 16)
            v = xl_v[l - l0, pl.ds(bg * BG + 16 * g, 16)]
            idxraw_v[buf, sl] = v
            idx2_v[buf, sl] = jnp.where(v >= HALF_V, v - HALF_V, v)
        pltpu.async_copy(tbl2.at[idx2_v.at[buf]], gbuf.at[buf], gs[buf])

    def wait_gather(buf):
        pltpu.make_async_copy(tbl2.at[idx2_v.at[buf]], gbuf.at[buf],
                              gs[buf]).wait()

    def fire_write(blk, buf):
        l = blk >> 3
        bg = blk & 7
        pltpu.async_copy(obuf.at[buf], out_jm.at[l, pl.ds(bg * BG, BG), :],
                         ws[buf])

    def wait_write(buf):
        pltpu.make_async_copy(obuf.at[buf], out_jm.at[0, pl.ds(0, BG), :],
                              ws[buf]).wait()

    def compute(blk, buf):
        l = blk >> 3
        pev = [pe_v[l, pl.ds(16 * c, 16)] for c in range(NCV)]
        ir = idxraw_v.at[buf]

        @pl.loop(0, BG)
        def _j(j):
            vspl = plsc.load_gather(ir, [jnp.full((16,), j, jnp.int32)])
            m = vspl >= HALF_V
            for c in range(NCV):
                a = gbuf[buf, j, pl.ds(16 * c, 16)]
                bb = gbuf[buf, j, pl.ds(DSZ + 16 * c, 16)]
                v = jnp.where(m, bb, a)
                obuf[buf, j, pl.ds(16 * c, 16)] = v * SCALE + pev[c]

    prep(base, 0)

    @pl.loop(0, PER_W, step=2)
    def _grp(j):
        for b in range(2):
            jj = j + b
            blk = base + jj
            wait_gather(b)

            @pl.when(jj + 1 < PER_W)
            def _():
                prep(blk + 1, 1 - b)

            @pl.when(jj >= 2)
            def _():
                wait_write(b)

            compute(blk, b)
            fire_write(blk, b)

    wait_write(0)
    wait_write(1)


def kernel(x, table):
    pe = _pos_encoding()
    xT = x.T
    tblT = table.T

    fmt = pl.pallas_call(
        _tc_fmt_body,
        grid=(FMT_G,),
        in_specs=[
            pl.BlockSpec((DSZ, FMT_B), lambda i: (0, i)),
            pl.BlockSpec((DSZ, FMT_B), lambda i: (0, i + FMT_G)),
        ],
        out_specs=pl.BlockSpec((FMT_B, 2 * DSZ), lambda i: (i, 0)),
        out_shape=jax.ShapeDtypeStruct((HALF_V, 2 * DSZ), jnp.float32),
    )
    tbl2 = fmt(tblT, tblT)

    run = pl.kernel(
        _sc_body,
        out_type=jax.ShapeDtypeStruct((L, B, DSZ), jnp.float32),
        mesh=plsc.VectorSubcoreMesh(core_axis_name="c", subcore_axis_name="s"),
        scratch_types=[
            pltpu.VMEM((L, DSZ), jnp.float32),
            pltpu.VMEM((16, B), jnp.int32),
            pltpu.VMEM((2, BG), jnp.int32),
            pltpu.VMEM((2, BG), jnp.int32),
            pltpu.VMEM((2, BG, 2 * DSZ), jnp.float32),
            pltpu.VMEM((2, BG, DSZ), jnp.float32),
        ]
        + [pltpu.SemaphoreType.DMA] * 4,
        compiler_params=pltpu.CompilerParams(
            use_tc_tiling_on_sc=True, needs_layout_passes=False
        ),
    )
    out_jm = run(xT, pe, tbl2)
    return out_jm.transpose(1, 0, 2)


# speedup vs baseline: 3.4087x; 3.4087x over previous
import math

import jax
import jax.numpy as jnp
from jax import lax
from jax.experimental import pallas as pl
from jax.experimental.pallas import tpu as pltpu
from jax.experimental.pallas import tpu_sc as plsc

VSZ = 1000000
DSZ = 64
MXLEN = 1000
MAX_TIMESCALE = 10000.0
B = 1024
L = 200

NC = 2
NS = 16
NW = NC * NS
BG = 128
NBG = B // BG
NBLK = L * NBG
PER_W = NBLK // NW
SCALE = math.sqrt(DSZ)
NG = BG // 16
NCV = DSZ // 16

NBK = (VSZ + 127) // 128
FMT_IT = (NBK + NW - 1) // NW
FMT_LOOP = ((FMT_IT + 2) + 3) // 4 * 4 + 4


def _pos_encoding():
    log_inc = math.log(MAX_TIMESCALE) / DSZ
    inv = jnp.exp(jnp.arange(0, DSZ, 2, dtype=jnp.float32) * -log_inc)
    pos = jnp.arange(0, MXLEN, dtype=jnp.float32)[:, None]
    pe = jnp.zeros((MXLEN, DSZ), jnp.float32)
    pe = pe.at[:, 0::2].set(jnp.sin(pos * inv))
    pe = pe.at[:, 1::2].set(jnp.cos(pos * inv))
    return pe[:L]


def _fmt_body(tblT, scratch, sbuf, tbuf, *sems):
    rs = sems[:4]
    ws = sems[4:]
    wid = lax.axis_index("s") * NC + lax.axis_index("c")
    iota = lax.iota(jnp.int32, 16)
    rds = [iota + 16 * d0 for d0 in range(4)]

    def blk_of(i):
        return wid + NW * i

    def fire_read(i, b):
        blk = blk_of(i)

        @pl.when(blk < NBK)
        def _():
            pltpu.async_copy(tblT.at[:, pl.ds(blk * 128, 128)], sbuf.at[b],
                             rs[b])

    def wait_read(i, b):
        @pl.when(blk_of(i) < NBK)
        def _():
            pltpu.make_async_copy(tblT.at[:, pl.ds(0, 128)], sbuf.at[b],
                                  rs[b]).wait()

    def fire_write(i, b):
        blk = blk_of(i)

        @pl.when(blk < NBK - 1)
        def _():
            pltpu.async_copy(tbuf.at[b], scratch.at[pl.ds(blk * 64, 64), :],
                             ws[b])

        @pl.when(blk == NBK - 1)
        def _():
            pltpu.async_copy(tbuf.at[b, pl.ds(0, 32), :],
                             scratch.at[pl.ds(blk * 64, 32), :], ws[b])

    def wait_write(i, b):
        blk = blk_of(i)

        @pl.when(blk < NBK - 1)
        def _():
            pltpu.make_async_copy(tbuf.at[b],
                                  scratch.at[pl.ds(0, 64), :], ws[b]).wait()

        @pl.when(blk == NBK - 1)
        def _():
            pltpu.make_async_copy(tbuf.at[b, pl.ds(0, 32), :],
                                  scratch.at[pl.ds(0, 32), :], ws[b]).wait()

    def transpose(i, b):
        @pl.when(blk_of(i) < NBK)
        def _():
            sb = sbuf.at[b]
            tb = tbuf.at[b]

            @pl.loop(0, 128, unroll=4)
            def _j(j0):
                cj = (jnp.full((16,), j0, jnp.int32) + iota) & 127
                pr = lax.shift_right_logical(cj, 1)
                pc0 = (cj & 1) * DSZ
                for d0 in range(4):
                    vals = plsc.load_gather(sb, [rds[d0], cj])
                    plsc.store_scatter(tb, [pr, pc0 + rds[d0]], vals)

    fire_read(0, 0)
    fire_read(1, 1)

    @pl.loop(0, FMT_LOOP, step=4)
    def _grp(i):
        for b in range(4):
            ii = i + b
            wait_read(ii, b)
            b2 = (b + 2) % 4

            @pl.when(ii >= 2)
            def _():
                wait_write(ii - 2, b2)

            fire_read(ii + 2, b2)
            transpose(ii, b)
            fire_write(ii, b)


def _sc_body(xT, pe_hbm, tbl2, out_jm,
             pe_v, xl_v, idxraw_v, idx2_v, gbuf, obuf, *sems):
    gs = sems[:2]
    ws = sems[2:]
    wid = lax.axis_index("s") * NC + lax.axis_index("c")
    base = wid * PER_W

    pltpu.sync_copy(pe_hbm, pe_v)
    l0 = pl.multiple_of(jnp.minimum((base >> 3) & ~7, L - 16), 8)
    pltpu.sync_copy(xT.at[pl.ds(l0, 16), :], xl_v)

    def prep(blk, buf):
        l = blk >> 3
        bg = blk & 7
        for g in range(NG):
            sl = pl.ds(16 * g, 16)
            v = xl_v[l - l0, pl.ds(bg * BG + 16 * g, 16)]
            idxraw_v[buf, sl] = v
            idx2_v[buf, sl] = lax.shift_right_logical(v, 1)
        pltpu.async_copy(tbl2.at[idx2_v.at[buf]], gbuf.at[buf], gs[buf])

    def wait_gather(buf):
        pltpu.make_async_copy(tbl2.at[idx2_v.at[buf]], gbuf.at[buf],
                              gs[buf]).wait()

    def fire_write(blk, buf):
        l = blk >> 3
        bg = blk & 7
        pltpu.async_copy(obuf.at[buf], out_jm.at[l, pl.ds(bg * BG, BG), :],
                         ws[buf])

    def wait_write(buf):
        pltpu.make_async_copy(obuf.at[buf], out_jm.at[0, pl.ds(0, BG), :],
                              ws[buf]).wait()

    def compute(blk, buf):
        l = blk >> 3
        pev = [pe_v[l, pl.ds(16 * c, 16)] for c in range(NCV)]
        ir = idxraw_v.at[buf]

        @pl.loop(0, BG)
        def _j(j):
            vspl = plsc.load_gather(ir, [jnp.full((16,), j, jnp.int32)])
            m = (vspl & 1) > 0
            for c in range(NCV):
                a = gbuf[buf, j, pl.ds(16 * c, 16)]
                bb = gbuf[buf, j, pl.ds(DSZ + 16 * c, 16)]
                v = jnp.where(m, bb, a)
                obuf[buf, j, pl.ds(16 * c, 16)] = v * SCALE + pev[c]

    prep(base, 0)

    @pl.loop(0, PER_W, step=2)
    def _grp(j):
        for b in range(2):
            jj = j + b
            blk = base + jj
            wait_gather(b)

            @pl.when(jj + 1 < PER_W)
            def _():
                prep(blk + 1, 1 - b)

            @pl.when(jj >= 2)
            def _():
                wait_write(b)

            compute(blk, b)
            fire_write(blk, b)

    wait_write(0)
    wait_write(1)


def kernel(x, table):
    pe = _pos_encoding()
    xT = x.T
    tblT = table.T

    fmt = pl.kernel(
        _fmt_body,
        out_type=jax.ShapeDtypeStruct((VSZ // 2, 2 * DSZ), jnp.float32),
        mesh=plsc.VectorSubcoreMesh(core_axis_name="c", subcore_axis_name="s"),
        scratch_types=[
            pltpu.VMEM((4, DSZ, 128), jnp.float32),
            pltpu.VMEM((4, DSZ, 2 * DSZ), jnp.float32),
        ]
        + [pltpu.SemaphoreType.DMA] * 8,
        compiler_params=pltpu.CompilerParams(
            use_tc_tiling_on_sc=True,
            needs_layout_passes=False,
            disable_bounds_checks=True,
        ),
    )
    tbl2 = fmt(tblT)

    run = pl.kernel(
        _sc_body,
        out_type=jax.ShapeDtypeStruct((L, B, DSZ), jnp.float32),
        mesh=plsc.VectorSubcoreMesh(core_axis_name="c", subcore_axis_name="s"),
        scratch_types=[
            pltpu.VMEM((L, DSZ), jnp.float32),
            pltpu.VMEM((16, B), jnp.int32),
            pltpu.VMEM((2, BG), jnp.int32),
            pltpu.VMEM((2, BG), jnp.int32),
            pltpu.VMEM((2, BG, 2 * DSZ), jnp.float32),
            pltpu.VMEM((2, BG, DSZ), jnp.float32),
        ]
        + [pltpu.SemaphoreType.DMA] * 4,
        compiler_params=pltpu.CompilerParams(
            use_tc_tiling_on_sc=True, needs_layout_passes=False
        ),
    )
    out_jm = run(xT, pe, tbl2)
    return out_jm.transpose(1, 0, 2)


# speedup vs baseline: 3.4177x; 1.0026x over previous
import math

import jax
import jax.numpy as jnp
from jax import lax
from jax.experimental import pallas as pl
from jax.experimental.pallas import tpu as pltpu
from jax.experimental.pallas import tpu_sc as plsc

VSZ = 1000000
DSZ = 64
MXLEN = 1000
MAX_TIMESCALE = 10000.0
B = 1024
L = 200

NC = 2
NS = 16
NW = NC * NS
BG = 128
NBG = B // BG
NBLK = L * NBG
PER_W = NBLK // NW
SCALE = math.sqrt(DSZ)
NG = BG // 16
NCV = DSZ // 16

NBK = (VSZ + 127) // 128
FMT_IT = (NBK + NW - 1) // NW
FMT_LOOP = ((FMT_IT + 2) + 3) // 4 * 4 + 4


def _pos_encoding():
    log_inc = math.log(MAX_TIMESCALE) / DSZ
    inv = jnp.exp(jnp.arange(0, DSZ, 2, dtype=jnp.float32) * -log_inc)
    pos = jnp.arange(0, MXLEN, dtype=jnp.float32)[:, None]
    pe = jnp.zeros((MXLEN, DSZ), jnp.float32)
    pe = pe.at[:, 0::2].set(jnp.sin(pos * inv))
    pe = pe.at[:, 1::2].set(jnp.cos(pos * inv))
    return pe[:L]


def _fmt_body(tblT, scratch, sbuf, tbuf, *sems):
    rs = sems[:4]
    ws = sems[4:]
    wid = lax.axis_index("s") * NC + lax.axis_index("c")
    iota = lax.iota(jnp.int32, 16)
    rds = [iota + 16 * d0 for d0 in range(4)]

    def blk_of(i):
        return wid + NW * i

    def fire_read(i, b):
        blk = blk_of(i)

        @pl.when(blk < NBK)
        def _():
            pltpu.async_copy(tblT.at[:, pl.ds(blk * 128, 128)], sbuf.at[b],
                             rs[b])

    def wait_read(i, b):
        @pl.when(blk_of(i) < NBK)
        def _():
            pltpu.make_async_copy(tblT.at[:, pl.ds(0, 128)], sbuf.at[b],
                                  rs[b]).wait()

    def fire_write(i, b):
        blk = blk_of(i)

        @pl.when(blk < NBK - 1)
        def _():
            pltpu.async_copy(tbuf.at[b], scratch.at[pl.ds(blk * 64, 64), :],
                             ws[b])

        @pl.when(blk == NBK - 1)
        def _():
            pltpu.async_copy(tbuf.at[b, pl.ds(0, 32), :],
                             scratch.at[pl.ds(blk * 64, 32), :], ws[b])

    def wait_write(i, b):
        blk = blk_of(i)

        @pl.when(blk < NBK - 1)
        def _():
            pltpu.make_async_copy(tbuf.at[b],
                                  scratch.at[pl.ds(0, 64), :], ws[b]).wait()

        @pl.when(blk == NBK - 1)
        def _():
            pltpu.make_async_copy(tbuf.at[b, pl.ds(0, 32), :],
                                  scratch.at[pl.ds(0, 32), :], ws[b]).wait()

    def transpose(i, b):
        @pl.when(blk_of(i) < NBK)
        def _():
            sb = sbuf.at[b]
            tb = tbuf.at[b]

            @pl.loop(0, 128, unroll=4)
            def _j(j0):
                cj = (jnp.full((16,), j0, jnp.int32) + iota) & 127
                pr = lax.shift_right_logical(cj, 1)
                pc0 = (cj & 1) * DSZ
                for d0 in range(4):
                    vals = plsc.load_gather(sb, [rds[d0], cj])
                    plsc.store_scatter(tb, [pr, pc0 + rds[d0]], vals)

    fire_read(0, 0)
    fire_read(1, 1)

    @pl.loop(0, FMT_LOOP, step=4)
    def _grp(i):
        for b in range(4):
            ii = i + b
            wait_read(ii, b)
            b2 = (b + 2) % 4

            @pl.when(ii >= 2)
            def _():
                wait_write(ii - 2, b2)

            fire_read(ii + 2, b2)
            transpose(ii, b)
            fire_write(ii, b)


def _sc_body(xT, pe_hbm, tbl2, out_jm,
             pe_v, xl_v, idxraw_v, idx2_v, gbuf, obuf, *sems):
    gs = sems[:2]
    ws = sems[2:]
    wid = lax.axis_index("s") * NC + lax.axis_index("c")
    base = wid * PER_W

    pltpu.sync_copy(pe_hbm, pe_v)
    l0 = pl.multiple_of(jnp.minimum((base >> 3) & ~7, L - 16), 8)
    pltpu.sync_copy(xT.at[pl.ds(l0, 16), :], xl_v)

    def prep(blk, buf):
        l = blk >> 3
        bg = blk & 7
        for g in range(NG):
            sl = pl.ds(16 * g, 16)
            v = xl_v[l - l0, pl.ds(bg * BG + 16 * g, 16)]
            idxraw_v[buf, sl] = v
            idx2_v[buf, sl] = lax.shift_right_logical(v, 1)
        pltpu.async_copy(tbl2.at[idx2_v.at[buf]], gbuf.at[buf], gs[buf])

    def wait_gather(buf):
        pltpu.make_async_copy(tbl2.at[idx2_v.at[buf]], gbuf.at[buf],
                              gs[buf]).wait()

    def fire_write(blk, buf):
        l = blk >> 3
        bg = blk & 7
        pltpu.async_copy(obuf.at[buf], out_jm.at[l, pl.ds(bg * BG, BG), :],
                         ws[buf])

    def wait_write(buf):
        pltpu.make_async_copy(obuf.at[buf], out_jm.at[0, pl.ds(0, BG), :],
                              ws[buf]).wait()

    def compute(blk, buf):
        l = blk >> 3
        pev = [pe_v[l, pl.ds(16 * c, 16)] for c in range(NCV)]
        ir = idxraw_v.at[buf]

        @pl.loop(0, BG, unroll=2)
        def _j(j):
            vspl = plsc.load_gather(ir, [jnp.full((16,), j, jnp.int32)])
            m = (vspl & 1) > 0
            for c in range(NCV):
                a = gbuf[buf, j, pl.ds(16 * c, 16)]
                bb = gbuf[buf, j, pl.ds(DSZ + 16 * c, 16)]
                v = jnp.where(m, bb, a)
                obuf[buf, j, pl.ds(16 * c, 16)] = v * SCALE + pev[c]

    prep(base, 0)

    @pl.loop(0, PER_W, step=2)
    def _grp(j):
        for b in range(2):
            jj = j + b
            blk = base + jj
            wait_gather(b)

            @pl.when(jj + 1 < PER_W)
            def _():
                prep(blk + 1, 1 - b)

            @pl.when(jj >= 2)
            def _():
                wait_write(b)

            compute(blk, b)
            fire_write(blk, b)

    wait_write(0)
    wait_write(1)


def kernel(x, table):
    pe = _pos_encoding()
    xT = x.T
    tblT = table.T

    fmt = pl.kernel(
        _fmt_body,
        out_type=jax.ShapeDtypeStruct((VSZ // 2, 2 * DSZ), jnp.float32),
        mesh=plsc.VectorSubcoreMesh(core_axis_name="c", subcore_axis_name="s"),
        scratch_types=[
            pltpu.VMEM((4, DSZ, 128), jnp.float32),
            pltpu.VMEM((4, DSZ, 2 * DSZ), jnp.float32),
        ]
        + [pltpu.SemaphoreType.DMA] * 8,
        compiler_params=pltpu.CompilerParams(
            use_tc_tiling_on_sc=True,
            needs_layout_passes=False,
            disable_bounds_checks=True,
        ),
    )
    tbl2 = fmt(tblT)

    run = pl.kernel(
        _sc_body,
        out_type=jax.ShapeDtypeStruct((L, B, DSZ), jnp.float32),
        mesh=plsc.VectorSubcoreMesh(core_axis_name="c", subcore_axis_name="s"),
        scratch_types=[
            pltpu.VMEM((L, DSZ), jnp.float32),
            pltpu.VMEM((16, B), jnp.int32),
            pltpu.VMEM((2, BG), jnp.int32),
            pltpu.VMEM((2, BG), jnp.int32),
            pltpu.VMEM((2, BG, 2 * DSZ), jnp.float32),
            pltpu.VMEM((2, BG, DSZ), jnp.float32),
        ]
        + [pltpu.SemaphoreType.DMA] * 4,
        compiler_params=pltpu.CompilerParams(
            use_tc_tiling_on_sc=True, needs_layout_passes=False
        ),
    )
    out_jm = run(xT, pe, tbl2)
    return out_jm.transpose(1, 0, 2)
